# Initial kernel scaffold; baseline (speedup 1.0000x reference)
import functools

import jax
import jax.numpy as jnp
from jax import lax
from jax.experimental import pallas as pl
from jax.experimental.pallas import tpu as pltpu
from jax.experimental.pallas import tpu_sc as plsc

N = 10000
E = 320000
D = 128

NC = 2
NS = 16
L = 16
NW = NC * NS

CH = 80
NB = 5
CPW = E // (NW * CH)
ROWS_PER_TILE = N // NS
ZCH = 25

_mesh = plsc.VectorSubcoreMesh(core_axis_name="c", subcore_axis_name="s")

_SPLAT_DNUMS = lax.GatherDimensionNumbers(
    offset_dims=(), collapsed_slice_dims=(0,), start_index_map=(0,))


def _splat(vec, lane):
    idx = jnp.full((L, 1), lane, jnp.int32)
    return lax.gather(vec, idx, _SPLAT_DNUMS, (1,),
                      mode=lax.GatherScatterMode.PROMISE_IN_BOUNDS)


@functools.partial(
    pl.kernel,
    out_type=jax.ShapeDtypeStruct((NC, N, D), jnp.float32),
    mesh=_mesh,
    scratch_types=[
        pltpu.VMEM((CPW, CH), jnp.int32),
        pltpu.VMEM((CPW, CH), jnp.int32),
        pltpu.VMEM((CPW, CH), jnp.float32),
        pltpu.VMEM_SHARED((N, D), jnp.float32),
        pltpu.VMEM((ZCH, D), jnp.float32),
    ]
    + [pltpu.VMEM((CH, D), jnp.float32) for _ in range(NB)]
    + [pltpu.SemaphoreType.DMA for _ in range(NB)],
)
def _sc_agg(sup_hbm, col_hbm, row_hbm, w_hbm, out_hbm,
            col_v, row_v, w_v, acc, zbuf, *rest):
    rbufs = rest[:NB]
    gsems = rest[NB:]
    cid = lax.axis_index("c")
    sid = lax.axis_index("s")
    wid = cid * NS + sid
    cbase = wid * CPW

    pltpu.sync_copy(col_hbm.at[pl.ds(cbase, CPW)], col_v)
    pltpu.sync_copy(row_hbm.at[pl.ds(cbase, CPW)], row_v)
    pltpu.sync_copy(w_hbm.at[pl.ds(cbase, CPW)], w_v)

    zero = jnp.zeros((L,), jnp.float32)
    for r in range(ZCH):
        for q in range(D // L):
            zbuf[r, pl.ds(q * L, L)] = zero
    rbase = sid * ROWS_PER_TILE

    @pl.loop(0, ROWS_PER_TILE // ZCH)
    def _(k):
        pltpu.sync_copy(zbuf, acc.at[pl.ds(rbase + k * ZCH, ZCH)])

    plsc.subcore_barrier()

    for b in range(NB - 1):
        pltpu.async_copy(sup_hbm.at[col_v.at[b]], rbufs[b], gsems[b])

    @pl.loop(0, CPW // NB)
    def _(p):
        i0 = p * NB
        for b in range(NB):
            i = i0 + b
            nb_ = (b + NB - 1) % NB
            ni = i + NB - 1

            @pl.when(ni < CPW)
            def _():
                pltpu.async_copy(sup_hbm.at[col_v.at[ni]], rbufs[nb_],
                                 gsems[nb_])

            pltpu.make_async_copy(sup_hbm.at[col_v.at[i]], rbufs[b],
                                  gsems[b]).wait()

            rbuf = rbufs[b]

            @pl.loop(0, CH // L)
            def _(g):
                wvec = w_v[i, pl.ds(g * L, L)]
                for l in range(L):
                    sp = _splat(wvec, l)
                    e = g * L + l
                    for q in range(D // L):
                        sl = pl.ds(q * L, L)
                        rbuf[e, sl] = rbuf[e, sl] * sp

            pltpu.sync_copy(rbuf, acc.at[row_v.at[i]], add=True)

    plsc.subcore_barrier()

    @pl.loop(0, ROWS_PER_TILE // ZCH)
    def _(k):
        off = rbase + k * ZCH
        pltpu.sync_copy(acc.at[pl.ds(off, ZCH)],
                        out_hbm.at[cid, pl.ds(off, ZCH)])


BLK = 2000


def _mm1_body(x_ref, w_ref, o_ref):
    o_ref[...] = jnp.dot(x_ref[...], w_ref[...],
                         preferred_element_type=jnp.float32,
                         precision=lax.Precision.HIGHEST)


def _mid_body(p_ref, b_ref, w_ref, o_ref):
    h = p_ref[0] + p_ref[1] + b_ref[...]
    h = jnp.maximum(h, 0.0)
    o_ref[...] = jnp.dot(h, w_ref[...],
                         preferred_element_type=jnp.float32,
                         precision=lax.Precision.HIGHEST)


def _fin_body(p_ref, b_ref, o_ref):
    o_ref[...] = p_ref[0] + p_ref[1] + b_ref[...]


_mm1 = pl.pallas_call(
    _mm1_body,
    grid=(N // BLK,),
    in_specs=[
        pl.BlockSpec((BLK, D), lambda i: (i, 0)),
        pl.BlockSpec((D, D), lambda i: (0, 0)),
    ],
    out_specs=pl.BlockSpec((BLK, D), lambda i: (i, 0)),
    out_shape=jax.ShapeDtypeStruct((N, D), jnp.float32),
)

_mid = pl.pallas_call(
    _mid_body,
    grid=(N // BLK,),
    in_specs=[
        pl.BlockSpec((NC, BLK, D), lambda i: (0, i, 0)),
        pl.BlockSpec((1, D), lambda i: (0, 0)),
        pl.BlockSpec((D, D), lambda i: (0, 0)),
    ],
    out_specs=pl.BlockSpec((BLK, D), lambda i: (i, 0)),
    out_shape=jax.ShapeDtypeStruct((N, D), jnp.float32),
)

_fin = pl.pallas_call(
    _fin_body,
    grid=(N // BLK,),
    in_specs=[
        pl.BlockSpec((NC, BLK, D), lambda i: (0, i, 0)),
        pl.BlockSpec((1, D), lambda i: (0, 0)),
    ],
    out_specs=pl.BlockSpec((BLK, D), lambda i: (i, 0)),
    out_shape=jax.ShapeDtypeStruct((N, D), jnp.float32),
)


@jax.jit
def kernel(x, edge_index, edge_weight, W1, b1, W2, b2):
    ei = edge_index.astype(jnp.int32)
    row2 = ei[0].reshape(E // CH, CH)
    col2 = ei[1].reshape(E // CH, CH)
    w2d = edge_weight.reshape(E // CH, CH)
    b1r = b1.reshape(1, D)
    b2r = b2.reshape(1, D)

    s1 = _mm1(x, W1)
    p = _sc_agg(s1, col2, row2, w2d)
    s2 = _mid(p, b1r, W2)
    q = _sc_agg(s2, col2, row2, w2d)
    return _fin(q, b2r)


# speedup vs baseline: 8.1009x; 8.1009x over previous
import dataclasses
import functools

import jax
import jax.numpy as jnp
from jax import lax
from jax.experimental import pallas as pl
from jax.experimental.pallas import tpu as pltpu
from jax.experimental.pallas import tpu_sc as plsc

N = 10000
E = 320000
D = 128

NC = 2
NS = 16
L = 16
NW = NC * NS

CH = 80
NB = 4
CNB = 8
CPW = E // (NW * CH)
ROUNDS = (CPW + CNB - 1) // CNB
ZCH = 8
NZCH = N // ZCH
ZPT = (NZCH + NS - 1) // NS

_SPLAT_DNUMS = lax.GatherDimensionNumbers(
    offset_dims=(), collapsed_slice_dims=(0,), start_index_map=(0,))


def _splat(vec, lane):
    idx = jnp.full((L, 1), lane, jnp.int32)
    return lax.gather(vec, idx, _SPLAT_DNUMS, (1,),
                      mode=lax.GatherScatterMode.PROMISE_IN_BOUNDS)


def _sc_agg_body(sup_hbm, crw_hbm, out_hbm, acc, zbuf, *rest):
    cbufs = rest[:CNB]
    rbufs = rest[CNB:CNB + NB]
    csems = rest[CNB + NB:2 * CNB + NB]
    gsems = rest[2 * CNB + NB:]
    cid = lax.axis_index("c")
    sid = lax.axis_index("s")
    wid = cid * NS + sid

    zero = jnp.zeros((L,), jnp.float32)
    for r in range(ZCH):
        for q in range(D // L):
            zbuf[r, pl.ds(q * L, L)] = zero

    @pl.loop(0, ZPT)
    def _(k):
        c8 = k * NS + sid

        @pl.when(c8 < NZCH)
        def _():
            pltpu.sync_copy(zbuf, acc.at[pl.ds(c8 * ZCH, ZCH)])

    plsc.subcore_barrier()

    for c in range(CNB):
        pltpu.async_copy(crw_hbm.at[wid, c], cbufs[c], csems[c])
    for c in range(NB - 1):
        pltpu.make_async_copy(crw_hbm.at[wid, c], cbufs[c], csems[c]).wait()
        pltpu.async_copy(sup_hbm.at[cbufs[c].at[0]], rbufs[c], gsems[c])

    @pl.loop(0, ROUNDS)
    def _(p):
        i0 = p * CNB
        for b in range(CNB):
            i = i0 + b
            rb = b % NB
            nrb = (b + NB - 1) % NB
            ncb = (b + NB - 1) % CNB
            ni = i + NB - 1

            @pl.when(ni < CPW)
            def _():
                pltpu.make_async_copy(crw_hbm.at[wid, ni], cbufs[ncb],
                                      csems[ncb]).wait()
                pltpu.async_copy(sup_hbm.at[cbufs[ncb].at[0]], rbufs[nrb],
                                 gsems[nrb])

            @pl.when(i < CPW)
            def _():
                pltpu.make_async_copy(sup_hbm.at[cbufs[b].at[0]], rbufs[rb],
                                      gsems[rb]).wait()

                rbuf = rbufs[rb]
                cbuf = cbufs[b]

                @pl.loop(0, CH // L)
                def _(g):
                    wvec = plsc.bitcast(cbuf[2, pl.ds(g * L, L)], jnp.float32)
                    for l in range(L):
                        sp = _splat(wvec, l)
                        e = g * L + l
                        for q in range(D // L):
                            sl = pl.ds(q * L, L)
                            rbuf[e, sl] = rbuf[e, sl] * sp

                pltpu.sync_copy(rbuf, acc.at[cbuf.at[1]], add=True)

                @pl.when(i + CNB < CPW)
                def _():
                    pltpu.async_copy(crw_hbm.at[wid, i + CNB], cbufs[b],
                                     csems[b])

    plsc.subcore_barrier()

    @pl.loop(0, ZPT)
    def _(k):
        c8 = k * NS + sid

        @pl.when(c8 < NZCH)
        def _():
            off = c8 * ZCH
            pltpu.sync_copy(acc.at[pl.ds(off, ZCH)],
                            out_hbm.at[cid, pl.ds(off, ZCH)])


@functools.cache
def _get_sc_agg():
    mesh = plsc.VectorSubcoreMesh(core_axis_name="c", subcore_axis_name="s",
                                  num_cores=NC, num_subcores=NS)
    cp = pltpu.CompilerParams()
    if "needs_layout_passes" in pltpu.CompilerParams.__dataclass_fields__:
        cp = dataclasses.replace(cp, needs_layout_passes=False)
    return pl.kernel(
        _sc_agg_body,
        out_type=jax.ShapeDtypeStruct((NC, N, D), jnp.float32),
        mesh=mesh,
        compiler_params=cp,
        scratch_types=[
            pltpu.VMEM_SHARED((N, D), jnp.float32),
            pltpu.VMEM((ZCH, D), jnp.float32),
        ]
        + [pltpu.VMEM((3, CH), jnp.int32) for _ in range(CNB)]
        + [pltpu.VMEM((CH, D), jnp.float32) for _ in range(NB)]
        + [pltpu.SemaphoreType.DMA for _ in range(CNB)]
        + [pltpu.SemaphoreType.DMA for _ in range(NB)],
    )


BLK = 2000


def _mm1_body(x_ref, w_ref, o_ref):
    o_ref[...] = jnp.dot(x_ref[...], w_ref[...],
                         preferred_element_type=jnp.float32,
                         precision=lax.Precision.HIGHEST)


def _mid_body(p_ref, b_ref, w_ref, o_ref):
    h = p_ref[0] + p_ref[1] + b_ref[...]
    h = jnp.maximum(h, 0.0)
    o_ref[...] = jnp.dot(h, w_ref[...],
                         preferred_element_type=jnp.float32,
                         precision=lax.Precision.HIGHEST)


def _fin_body(p_ref, b_ref, o_ref):
    o_ref[...] = p_ref[0] + p_ref[1] + b_ref[...]


_mm1 = pl.pallas_call(
    _mm1_body,
    grid=(N // BLK,),
    in_specs=[
        pl.BlockSpec((BLK, D), lambda i: (i, 0)),
        pl.BlockSpec((D, D), lambda i: (0, 0)),
    ],
    out_specs=pl.BlockSpec((BLK, D), lambda i: (i, 0)),
    out_shape=jax.ShapeDtypeStruct((N, D), jnp.float32),
)

_mid = pl.pallas_call(
    _mid_body,
    grid=(N // BLK,),
    in_specs=[
        pl.BlockSpec((NC, BLK, D), lambda i: (0, i, 0)),
        pl.BlockSpec((1, D), lambda i: (0, 0)),
        pl.BlockSpec((D, D), lambda i: (0, 0)),
    ],
    out_specs=pl.BlockSpec((BLK, D), lambda i: (i, 0)),
    out_shape=jax.ShapeDtypeStruct((N, D), jnp.float32),
)

_fin = pl.pallas_call(
    _fin_body,
    grid=(N // BLK,),
    in_specs=[
        pl.BlockSpec((NC, BLK, D), lambda i: (0, i, 0)),
        pl.BlockSpec((1, D), lambda i: (0, 0)),
    ],
    out_specs=pl.BlockSpec((BLK, D), lambda i: (i, 0)),
    out_shape=jax.ShapeDtypeStruct((N, D), jnp.float32),
)


@jax.jit
def kernel(x, edge_index, edge_weight, W1, b1, W2, b2):
    ei = edge_index.astype(jnp.int32)
    col3 = ei[1].reshape(NW, CPW, 1, CH)
    row3 = ei[0].reshape(NW, CPW, 1, CH)
    wbits = lax.bitcast_convert_type(edge_weight, jnp.int32)
    w3 = wbits.reshape(NW, CPW, 1, CH)
    crw = jnp.concatenate([col3, row3, w3], axis=2)
    b1r = b1.reshape(1, D)
    b2r = b2.reshape(1, D)

    sc_agg = _get_sc_agg()
    s1 = _mm1(x, W1)
    p = sc_agg(s1, crw)
    s2 = _mid(p, b1r, W2)
    q = sc_agg(s2, crw)
    return _fin(q, b2r)


# speedup vs baseline: 8.3616x; 1.0322x over previous
import dataclasses
import functools

import jax
import jax.numpy as jnp
from jax import lax
from jax.experimental import pallas as pl
from jax.experimental.pallas import tpu as pltpu
from jax.experimental.pallas import tpu_sc as plsc

N = 10000
E = 320000
D = 128

NC = 2
NS = 16
L = 16
NW = NC * NS

CH = 80
NB = 4
CNB = 8
GA = 2
CPW = E // (NW * CH)
ROUNDS = (CPW + CNB - 1) // CNB
ZCH = 8
NZCH = N // ZCH
ZPT = (NZCH + NS - 1) // NS

_SPLAT_DNUMS = lax.GatherDimensionNumbers(
    offset_dims=(), collapsed_slice_dims=(0,), start_index_map=(0,))


def _splat(vec, lane):
    idx = jnp.full((L, 1), lane, jnp.int32)
    return lax.gather(vec, idx, _SPLAT_DNUMS, (1,),
                      mode=lax.GatherScatterMode.PROMISE_IN_BOUNDS)


def _sc_agg_body(sup_hbm, crw_hbm, out_hbm, acc, zbuf, *rest):
    cbufs = rest[:CNB]
    rbufs = rest[CNB:CNB + NB]
    csems = rest[CNB + NB:2 * CNB + NB]
    gsems = rest[2 * CNB + NB:2 * CNB + 2 * NB]
    ssems = rest[2 * CNB + 2 * NB:]
    cid = lax.axis_index("c")
    sid = lax.axis_index("s")
    wid = cid * NS + sid

    zero = jnp.zeros((L,), jnp.float32)
    for r in range(ZCH):
        for q in range(D // L):
            zbuf[r, pl.ds(q * L, L)] = zero

    @pl.loop(0, ZPT)
    def _(k):
        c8 = k * NS + sid

        @pl.when(c8 < NZCH)
        def _():
            pltpu.sync_copy(zbuf, acc.at[pl.ds(c8 * ZCH, ZCH)])

    plsc.subcore_barrier()

    for c in range(CNB):
        pltpu.async_copy(crw_hbm.at[wid, c], cbufs[c], csems[c])
    for c in range(GA):
        pltpu.make_async_copy(crw_hbm.at[wid, c], cbufs[c], csems[c]).wait()
        pltpu.async_copy(sup_hbm.at[cbufs[c].at[0]], rbufs[c], gsems[c])

    @pl.loop(0, ROUNDS)
    def _(p):
        i0 = p * CNB
        for b in range(CNB):
            i = i0 + b
            rb = b % NB
            pb = (b + GA) % NB
            pc = (b - GA) % CNB
            nc = (b + GA) % CNB

            @pl.when(jnp.logical_and(GA <= i, i < CPW + GA))
            def _():
                pltpu.make_async_copy(rbufs[pb], acc.at[cbufs[pc].at[1]],
                                      ssems[pb]).wait()

            @pl.when(jnp.logical_and(GA <= i, i + CNB - GA < CPW))
            def _():
                pltpu.async_copy(crw_hbm.at[wid, i + CNB - GA], cbufs[pc],
                                 csems[pc])

            @pl.when(i + GA < CPW)
            def _():
                pltpu.make_async_copy(crw_hbm.at[wid, i + GA], cbufs[nc],
                                      csems[nc]).wait()
                pltpu.async_copy(sup_hbm.at[cbufs[nc].at[0]], rbufs[pb],
                                 gsems[pb])

            @pl.when(i < CPW)
            def _():
                pltpu.make_async_copy(sup_hbm.at[cbufs[b].at[0]], rbufs[rb],
                                      gsems[rb]).wait()

                rbuf = rbufs[rb]
                cbuf = cbufs[b]

                @pl.loop(0, CH // L)
                def _(g):
                    wvec = plsc.bitcast(cbuf[2, pl.ds(g * L, L)], jnp.float32)
                    for l in range(L):
                        sp = _splat(wvec, l)
                        e = g * L + l
                        for q in range(D // L):
                            sl = pl.ds(q * L, L)
                            rbuf[e, sl] = rbuf[e, sl] * sp

                pltpu.async_copy(rbuf, acc.at[cbuf.at[1]], ssems[rb],
                                 add=True)

    plsc.subcore_barrier()

    @pl.loop(0, ZPT)
    def _(k):
        c8 = k * NS + sid

        @pl.when(c8 < NZCH)
        def _():
            off = c8 * ZCH
            pltpu.sync_copy(acc.at[pl.ds(off, ZCH)],
                            out_hbm.at[cid, pl.ds(off, ZCH)])


@functools.cache
def _get_sc_agg():
    mesh = plsc.VectorSubcoreMesh(core_axis_name="c", subcore_axis_name="s",
                                  num_cores=NC, num_subcores=NS)
    cp = pltpu.CompilerParams()
    if "needs_layout_passes" in pltpu.CompilerParams.__dataclass_fields__:
        cp = dataclasses.replace(cp, needs_layout_passes=False)
    return pl.kernel(
        _sc_agg_body,
        out_type=jax.ShapeDtypeStruct((NC, N, D), jnp.float32),
        mesh=mesh,
        compiler_params=cp,
        scratch_types=[
            pltpu.VMEM_SHARED((N, D), jnp.float32),
            pltpu.VMEM((ZCH, D), jnp.float32),
        ]
        + [pltpu.VMEM((3, CH), jnp.int32) for _ in range(CNB)]
        + [pltpu.VMEM((CH, D), jnp.float32) for _ in range(NB)]
        + [pltpu.SemaphoreType.DMA for _ in range(CNB)]
        + [pltpu.SemaphoreType.DMA for _ in range(2 * NB)],
    )


BLK = 2000


def _mm1_body(x_ref, w_ref, o_ref):
    o_ref[...] = jnp.dot(x_ref[...], w_ref[...],
                         preferred_element_type=jnp.float32,
                         precision=lax.Precision.HIGHEST)


def _mid_body(p_ref, b_ref, w_ref, o_ref):
    h = p_ref[0] + p_ref[1] + b_ref[...]
    h = jnp.maximum(h, 0.0)
    o_ref[...] = jnp.dot(h, w_ref[...],
                         preferred_element_type=jnp.float32,
                         precision=lax.Precision.HIGHEST)


def _fin_body(p_ref, b_ref, o_ref):
    o_ref[...] = p_ref[0] + p_ref[1] + b_ref[...]


_mm1 = pl.pallas_call(
    _mm1_body,
    grid=(N // BLK,),
    in_specs=[
        pl.BlockSpec((BLK, D), lambda i: (i, 0)),
        pl.BlockSpec((D, D), lambda i: (0, 0)),
    ],
    out_specs=pl.BlockSpec((BLK, D), lambda i: (i, 0)),
    out_shape=jax.ShapeDtypeStruct((N, D), jnp.float32),
)

_mid = pl.pallas_call(
    _mid_body,
    grid=(N // BLK,),
    in_specs=[
        pl.BlockSpec((NC, BLK, D), lambda i: (0, i, 0)),
        pl.BlockSpec((1, D), lambda i: (0, 0)),
        pl.BlockSpec((D, D), lambda i: (0, 0)),
    ],
    out_specs=pl.BlockSpec((BLK, D), lambda i: (i, 0)),
    out_shape=jax.ShapeDtypeStruct((N, D), jnp.float32),
)

_fin = pl.pallas_call(
    _fin_body,
    grid=(N // BLK,),
    in_specs=[
        pl.BlockSpec((NC, BLK, D), lambda i: (0, i, 0)),
        pl.BlockSpec((1, D), lambda i: (0, 0)),
    ],
    out_specs=pl.BlockSpec((BLK, D), lambda i: (i, 0)),
    out_shape=jax.ShapeDtypeStruct((N, D), jnp.float32),
)


@jax.jit
def kernel(x, edge_index, edge_weight, W1, b1, W2, b2):
    ei = edge_index.astype(jnp.int32)
    col3 = ei[1].reshape(NW, CPW, 1, CH)
    row3 = ei[0].reshape(NW, CPW, 1, CH)
    wbits = lax.bitcast_convert_type(edge_weight, jnp.int32)
    w3 = wbits.reshape(NW, CPW, 1, CH)
    crw = jnp.concatenate([col3, row3, w3], axis=2)
    b1r = b1.reshape(1, D)
    b2r = b2.reshape(1, D)

    sc_agg = _get_sc_agg()
    s1 = _mm1(x, W1)
    p = sc_agg(s1, crw)
    s2 = _mid(p, b1r, W2)
    q = sc_agg(s2, crw)
    return _fin(q, b2r)


# speedup vs baseline: 8.3639x; 1.0003x over previous
import dataclasses
import functools

import jax
import jax.numpy as jnp
from jax import lax
from jax.experimental import pallas as pl
from jax.experimental.pallas import tpu as pltpu
from jax.experimental.pallas import tpu_sc as plsc

N = 10000
E = 320000
D = 128

NC = 2
NS = 16
L = 16
NW = NC * NS

CH = 80
NB = 4
CNB = 8
GA = 2
CPW = E // (NW * CH)
ROUNDS = (CPW + CNB - 1) // CNB
ZCH = 8
NZCH = N // ZCH
ZPT = (NZCH + NS - 1) // NS

_SPLAT_DNUMS = lax.GatherDimensionNumbers(
    offset_dims=(), collapsed_slice_dims=(0,), start_index_map=(0,))


def _splat(vec, lane):
    idx = jnp.full((L, 1), lane, jnp.int32)
    return lax.gather(vec, idx, _SPLAT_DNUMS, (1,),
                      mode=lax.GatherScatterMode.PROMISE_IN_BOUNDS)


def _sc_agg_body(sup_hbm, crw_hbm, out_hbm, acc, zbuf, *rest):
    cbufs = rest[:CNB]
    rbufs = rest[CNB:CNB + NB]
    csems = rest[CNB + NB:2 * CNB + NB]
    gsems = rest[2 * CNB + NB:2 * CNB + 2 * NB]
    ssems = rest[2 * CNB + 2 * NB:]
    cid = lax.axis_index("c")
    sid = lax.axis_index("s")
    wid = cid * NS + sid

    zero = jnp.zeros((L,), jnp.float32)
    for r in range(ZCH):
        for q in range(D // L):
            zbuf[r, pl.ds(q * L, L)] = zero

    @pl.loop(0, ZPT)
    def _(k):
        c8 = k * NS + sid

        @pl.when(c8 < NZCH)
        def _():
            pltpu.sync_copy(zbuf, acc.at[pl.ds(c8 * ZCH, ZCH)])

    plsc.subcore_barrier()

    for c in range(CNB):
        pltpu.async_copy(crw_hbm.at[wid, c], cbufs[c], csems[c])
    for c in range(GA):
        pltpu.make_async_copy(crw_hbm.at[wid, c], cbufs[c], csems[c]).wait()
        pltpu.async_copy(sup_hbm.at[cbufs[c].at[0]], rbufs[c], gsems[c])

    @pl.loop(0, ROUNDS)
    def _(p):
        i0 = p * CNB
        for b in range(CNB):
            i = i0 + b
            rb = b % NB
            nrb = (b + GA) % NB
            pc = (b - GA) % CNB
            nc = (b + GA) % CNB

            @pl.when(jnp.logical_and(GA <= i, i < CPW + GA))
            def _():
                pltpu.make_async_copy(rbufs[nrb], acc.at[cbufs[pc].at[1]],
                                      ssems[nrb]).wait()

            @pl.when(jnp.logical_and(GA <= i, i + CNB - GA < CPW))
            def _():
                pltpu.async_copy(crw_hbm.at[wid, i + CNB - GA], cbufs[pc],
                                 csems[pc])

            @pl.when(i + GA < CPW)
            def _():
                pltpu.make_async_copy(crw_hbm.at[wid, i + GA], cbufs[nc],
                                      csems[nc]).wait()
                pltpu.async_copy(sup_hbm.at[cbufs[nc].at[0]], rbufs[nrb],
                                 gsems[nrb])

            @pl.when(i < CPW)
            def _():
                pltpu.make_async_copy(sup_hbm.at[cbufs[b].at[0]], rbufs[rb],
                                      gsems[rb]).wait()

                rbuf = rbufs[rb]
                cbuf = cbufs[b]

                @pl.loop(0, CH // L)
                def _(g):
                    wvec = plsc.bitcast(cbuf[2, pl.ds(g * L, L)], jnp.float32)
                    for l in range(L):
                        sp = _splat(wvec, l)
                        e = g * L + l
                        for q in range(D // L):
                            sl = pl.ds(q * L, L)
                            rbuf[e, sl] = rbuf[e, sl] * sp

                pltpu.async_copy(rbuf, acc.at[cbuf.at[1]], ssems[rb],
                                 add=True)

    plsc.subcore_barrier()

    @pl.loop(0, ZPT)
    def _(k):
        c8 = k * NS + sid

        @pl.when(c8 < NZCH)
        def _():
            off = c8 * ZCH
            pltpu.sync_copy(acc.at[pl.ds(off, ZCH)],
                            out_hbm.at[cid, pl.ds(off, ZCH)])


@functools.cache
def _get_sc_agg():
    mesh = plsc.VectorSubcoreMesh(core_axis_name="c", subcore_axis_name="s",
                                  num_cores=NC, num_subcores=NS)
    cp = pltpu.CompilerParams()
    if "needs_layout_passes" in pltpu.CompilerParams.__dataclass_fields__:
        cp = dataclasses.replace(cp, needs_layout_passes=False)
    return pl.kernel(
        _sc_agg_body,
        out_type=jax.ShapeDtypeStruct((NC, N, D), jnp.float32),
        mesh=mesh,
        compiler_params=cp,
        scratch_types=[
            pltpu.VMEM_SHARED((N, D), jnp.float32),
            pltpu.VMEM((ZCH, D), jnp.float32),
        ]
        + [pltpu.VMEM((3, CH), jnp.int32) for _ in range(CNB)]
        + [pltpu.VMEM((CH, D), jnp.float32) for _ in range(NB)]
        + [pltpu.SemaphoreType.DMA for _ in range(CNB)]
        + [pltpu.SemaphoreType.DMA for _ in range(2 * NB)],
    )


BLK = 2000


def _mm1_body(x_ref, w_ref, o_ref):
    o_ref[...] = jnp.dot(x_ref[...], w_ref[...],
                         preferred_element_type=jnp.float32,
                         precision=lax.Precision.HIGHEST)


def _mid_body(p_ref, b_ref, w_ref, o_ref):
    h = p_ref[0] + p_ref[1] + b_ref[...]
    h = jnp.maximum(h, 0.0)
    o_ref[...] = jnp.dot(h, w_ref[...],
                         preferred_element_type=jnp.float32,
                         precision=lax.Precision.HIGHEST)


def _fin_body(p_ref, b_ref, o_ref):
    o_ref[...] = p_ref[0] + p_ref[1] + b_ref[...]


_mm1 = pl.pallas_call(
    _mm1_body,
    grid=(N // BLK,),
    in_specs=[
        pl.BlockSpec((BLK, D), lambda i: (i, 0)),
        pl.BlockSpec((D, D), lambda i: (0, 0)),
    ],
    out_specs=pl.BlockSpec((BLK, D), lambda i: (i, 0)),
    out_shape=jax.ShapeDtypeStruct((N, D), jnp.float32),
)

_mid = pl.pallas_call(
    _mid_body,
    grid=(N // BLK,),
    in_specs=[
        pl.BlockSpec((NC, BLK, D), lambda i: (0, i, 0)),
        pl.BlockSpec((1, D), lambda i: (0, 0)),
        pl.BlockSpec((D, D), lambda i: (0, 0)),
    ],
    out_specs=pl.BlockSpec((BLK, D), lambda i: (i, 0)),
    out_shape=jax.ShapeDtypeStruct((N, D), jnp.float32),
)

_fin = pl.pallas_call(
    _fin_body,
    grid=(N // BLK,),
    in_specs=[
        pl.BlockSpec((NC, BLK, D), lambda i: (0, i, 0)),
        pl.BlockSpec((1, D), lambda i: (0, 0)),
    ],
    out_specs=pl.BlockSpec((BLK, D), lambda i: (i, 0)),
    out_shape=jax.ShapeDtypeStruct((N, D), jnp.float32),
)


@jax.jit
def kernel(x, edge_index, edge_weight, W1, b1, W2, b2):
    ei = edge_index.astype(jnp.int32)
    col3 = ei[1].reshape(NW, CPW, 1, CH)
    row3 = ei[0].reshape(NW, CPW, 1, CH)
    wbits = lax.bitcast_convert_type(edge_weight, jnp.int32)
    w3 = wbits.reshape(NW, CPW, 1, CH)
    crw = jnp.concatenate([col3, row3, w3], axis=2)
    b1r = b1.reshape(1, D)
    b2r = b2.reshape(1, D)

    sc_agg = _get_sc_agg()
    s1 = _mm1(x, W1)
    p = sc_agg(s1, crw)
    s2 = _mid(p, b1r, W2)
    q = sc_agg(s2, crw)
    return _fin(q, b2r)


# speedup vs baseline: 8.4222x; 1.0070x over previous
import dataclasses
import functools

import jax
import jax.numpy as jnp
from jax import lax
from jax.experimental import pallas as pl
from jax.experimental.pallas import tpu as pltpu
from jax.experimental.pallas import tpu_sc as plsc

N = 10000
E = 320000
D = 128

NC = 2
NS = 16
L = 16
NW = NC * NS

CH = 80
NB = 4
CNB = 8
GA = 2
CPW = E // (NW * CH)
ROUNDS = (CPW + CNB - 1) // CNB
ZCH = 8
NZCH = N // ZCH
ZPT = (NZCH + NS - 1) // NS

_SPLAT_DNUMS = lax.GatherDimensionNumbers(
    offset_dims=(), collapsed_slice_dims=(0,), start_index_map=(0,))


def _splat(vec, lane):
    idx = jnp.full((L, 1), lane, jnp.int32)
    return lax.gather(vec, idx, _SPLAT_DNUMS, (1,),
                      mode=lax.GatherScatterMode.PROMISE_IN_BOUNDS)


def _sc_agg_body(sup_hbm, crw_hbm, out_hbm, acc, zbuf, *rest):
    cbufs = rest[:CNB]
    rbufs = rest[CNB:CNB + NB]
    csems = rest[CNB + NB:2 * CNB + NB]
    gsems = rest[2 * CNB + NB:2 * CNB + 2 * NB]
    ssems = rest[2 * CNB + 2 * NB:]
    cid = lax.axis_index("c")
    sid = lax.axis_index("s")
    wid = cid * NS + sid

    zero = jnp.zeros((L,), jnp.float32)
    for r in range(ZCH):
        for q in range(D // L):
            zbuf[r, pl.ds(q * L, L)] = zero

    @pl.loop(0, ZPT)
    def _(k):
        c8 = k * NS + sid

        @pl.when(c8 < NZCH)
        def _():
            pltpu.sync_copy(zbuf, acc.at[pl.ds(c8 * ZCH, ZCH)])

    plsc.subcore_barrier()

    for c in range(CNB):
        pltpu.async_copy(crw_hbm.at[wid, c], cbufs[c], csems[c])
    for c in range(GA):
        pltpu.make_async_copy(crw_hbm.at[wid, c], cbufs[c], csems[c]).wait()
        pltpu.async_copy(sup_hbm.at[cbufs[c].at[0]], rbufs[c], gsems[c])

    @pl.loop(0, ROUNDS)
    def _(p):
        i0 = p * CNB
        for b in range(CNB):
            i = i0 + b
            rb = b % NB
            nrb = (b + GA) % NB
            pc = (b - GA) % CNB
            nc = (b + GA) % CNB

            @pl.when(jnp.logical_and(GA <= i, i < CPW + GA))
            def _():
                pltpu.make_async_copy(rbufs[nrb], acc.at[cbufs[pc].at[1]],
                                      ssems[nrb]).wait()

            @pl.when(jnp.logical_and(GA <= i, i + CNB - GA < CPW))
            def _():
                pltpu.async_copy(crw_hbm.at[wid, i + CNB - GA], cbufs[pc],
                                 csems[pc])

            @pl.when(i + GA < CPW)
            def _():
                pltpu.make_async_copy(crw_hbm.at[wid, i + GA], cbufs[nc],
                                      csems[nc]).wait()
                pltpu.async_copy(sup_hbm.at[cbufs[nc].at[0]], rbufs[nrb],
                                 gsems[nrb])

            @pl.when(i < CPW)
            def _():
                pltpu.make_async_copy(sup_hbm.at[cbufs[b].at[0]], rbufs[rb],
                                      gsems[rb]).wait()

                rbuf = rbufs[rb]
                cbuf = cbufs[b]

                @pl.loop(0, CH // L)
                def _(g):
                    wvec = plsc.bitcast(cbuf[2, pl.ds(g * L, L)], jnp.float32)
                    for l in range(L):
                        sp = _splat(wvec, l)
                        e = g * L + l
                        for q in range(D // L):
                            sl = pl.ds(q * L, L)
                            rbuf[e, sl] = rbuf[e, sl] * sp

                pltpu.async_copy(rbuf, acc.at[cbuf.at[1]], ssems[rb],
                                 add=True)

    plsc.subcore_barrier()

    @pl.loop(0, ZPT)
    def _(k):
        c8 = k * NS + sid

        @pl.when(c8 < NZCH)
        def _():
            off = c8 * ZCH
            pltpu.sync_copy(acc.at[pl.ds(off, ZCH)],
                            out_hbm.at[cid, pl.ds(off, ZCH)])


@functools.cache
def _get_sc_agg():
    mesh = plsc.VectorSubcoreMesh(core_axis_name="c", subcore_axis_name="s",
                                  num_cores=NC, num_subcores=NS)
    cp = pltpu.CompilerParams()
    if "needs_layout_passes" in pltpu.CompilerParams.__dataclass_fields__:
        cp = dataclasses.replace(cp, needs_layout_passes=False)
    return pl.kernel(
        _sc_agg_body,
        out_type=jax.ShapeDtypeStruct((NC, N, D), jnp.float32),
        mesh=mesh,
        compiler_params=cp,
        scratch_types=[
            pltpu.VMEM_SHARED((N, D), jnp.float32),
            pltpu.VMEM((ZCH, D), jnp.float32),
        ]
        + [pltpu.VMEM((3, CH), jnp.int32) for _ in range(CNB)]
        + [pltpu.VMEM((CH, D), jnp.float32) for _ in range(NB)]
        + [pltpu.SemaphoreType.DMA for _ in range(CNB)]
        + [pltpu.SemaphoreType.DMA for _ in range(2 * NB)],
    )


BLK = 2000


def _mm1_body(x_ref, w_ref, o_ref):
    o_ref[...] = jnp.dot(x_ref[...], w_ref[...],
                         preferred_element_type=jnp.float32,
                         precision=lax.Precision.DEFAULT)


def _mid_body(p_ref, b_ref, w_ref, o_ref):
    h = p_ref[0] + p_ref[1] + b_ref[...]
    h = jnp.maximum(h, 0.0)
    o_ref[...] = jnp.dot(h, w_ref[...],
                         preferred_element_type=jnp.float32,
                         precision=lax.Precision.DEFAULT)


def _fin_body(p_ref, b_ref, o_ref):
    o_ref[...] = p_ref[0] + p_ref[1] + b_ref[...]


_mm1 = pl.pallas_call(
    _mm1_body,
    grid=(N // BLK,),
    in_specs=[
        pl.BlockSpec((BLK, D), lambda i: (i, 0)),
        pl.BlockSpec((D, D), lambda i: (0, 0)),
    ],
    out_specs=pl.BlockSpec((BLK, D), lambda i: (i, 0)),
    out_shape=jax.ShapeDtypeStruct((N, D), jnp.float32),
)

_mid = pl.pallas_call(
    _mid_body,
    grid=(N // BLK,),
    in_specs=[
        pl.BlockSpec((NC, BLK, D), lambda i: (0, i, 0)),
        pl.BlockSpec((1, D), lambda i: (0, 0)),
        pl.BlockSpec((D, D), lambda i: (0, 0)),
    ],
    out_specs=pl.BlockSpec((BLK, D), lambda i: (i, 0)),
    out_shape=jax.ShapeDtypeStruct((N, D), jnp.float32),
)

_fin = pl.pallas_call(
    _fin_body,
    grid=(N // BLK,),
    in_specs=[
        pl.BlockSpec((NC, BLK, D), lambda i: (0, i, 0)),
        pl.BlockSpec((1, D), lambda i: (0, 0)),
    ],
    out_specs=pl.BlockSpec((BLK, D), lambda i: (i, 0)),
    out_shape=jax.ShapeDtypeStruct((N, D), jnp.float32),
)


@jax.jit
def kernel(x, edge_index, edge_weight, W1, b1, W2, b2):
    ei = edge_index.astype(jnp.int32)
    col3 = ei[1].reshape(NW, CPW, 1, CH)
    row3 = ei[0].reshape(NW, CPW, 1, CH)
    wbits = lax.bitcast_convert_type(edge_weight, jnp.int32)
    w3 = wbits.reshape(NW, CPW, 1, CH)
    crw = jnp.concatenate([col3, row3, w3], axis=2)
    b1r = b1.reshape(1, D)
    b2r = b2.reshape(1, D)

    sc_agg = _get_sc_agg()
    s1 = _mm1(x, W1)
    p = sc_agg(s1, crw)
    s2 = _mid(p, b1r, W2)
    q = sc_agg(s2, crw)
    return _fin(q, b2r)
